# baseline (speedup 1.0000x reference)
import functools

import jax
import jax.numpy as jnp
from jax import lax
from jax.experimental import pallas as pl
from jax.experimental.pallas import tpu as pltpu
from jax.experimental.pallas import tpu_sc as plsc

_VOCAB = 100000
_NC, _NS, _L = 2, 16, 16
_NW = _NC * _NS
_RCHUNK = 40
_NBUF = 2
_K_SPM = 73600


def _gather_call(idx_t, vocab_table):
    rows, cols = idx_t.shape
    cb = cols // _NW
    nchunks = rows // _RCHUNK
    rpair = _RCHUNK // 2
    nv = cb // _L
    mesh = plsc.VectorSubcoreMesh(core_axis_name="c", subcore_axis_name="s")

    @functools.partial(
        pl.kernel,
        mesh=mesh,
        compiler_params=pltpu.CompilerParams(needs_layout_passes=False),
        out_type=jax.ShapeDtypeStruct((rows, cols), jnp.float32),
        scratch_types=[
            pltpu.VMEM_SHARED((_K_SPM,), jnp.float32),
            pltpu.VMEM((_VOCAB,), jnp.float32),
            pltpu.VMEM((_RCHUNK, cb), jnp.int32),
            pltpu.VMEM((_RCHUNK, cb), jnp.int32),
            pltpu.VMEM((_RCHUNK, cb), jnp.float32),
            pltpu.VMEM((_RCHUNK, cb), jnp.float32),
            pltpu.SemaphoreType.DMA,
            pltpu.SemaphoreType.DMA,
            pltpu.SemaphoreType.DMA,
            pltpu.SemaphoreType.DMA,
            pltpu.SemaphoreType.DMA,
            pltpu.SemaphoreType.DMA,
        ],
    )
    def k(table_hbm, idx_hbm, out_hbm, table_sh, table_v, idx_v0, idx_v1,
          out_v0, out_v1, sem_t, sem_t2, sem_i0, sem_i1, sem_o0, sem_o1):
        idx_b = (idx_v0, idx_v1)
        out_b = (out_v0, out_v1)
        sem_i = (sem_i0, sem_i1)
        sem_o = (sem_o0, sem_o1)
        nib = len(idx_b)
        wid = lax.axis_index("s") * _NC + lax.axis_index("c")
        col0 = wid * cb

        i_cp = [None] * nchunks
        o_cp = [None] * nchunks
        for c in range(min(nib, nchunks)):
            i_cp[c] = pltpu.async_copy(
                idx_hbm.at[pl.ds(c * _RCHUNK, _RCHUNK), pl.ds(col0, cb)],
                idx_b[c % nib], sem_i[c % nib])
        t2_cp = pltpu.async_copy(
            table_hbm.at[pl.ds(_K_SPM, _VOCAB - _K_SPM)],
            table_v.at[pl.ds(_K_SPM, _VOCAB - _K_SPM)], sem_t2)

        @pl.when(lax.axis_index("s") == 0)
        def _load_spmem():
            pltpu.sync_copy(table_hbm.at[pl.ds(0, _K_SPM)], table_sh)

        plsc.subcore_barrier()
        t_cp = pltpu.async_copy(table_sh, table_v.at[pl.ds(0, _K_SPM)], sem_t)
        t_cp.wait()
        t2_cp.wait()

        for c in range(nchunks):
            i_cp[c].wait()
            if c >= _NBUF:
                o_cp[c - _NBUF].wait()
            src = idx_b[c % nib]
            dst = out_b[c % _NBUF]

            def body(r, carry):
                locs = [(r * 2 + j, pl.ds(v * _L, _L))
                        for j in range(2) for v in range(nv)]
                idxs = [src[rr, sl] for rr, sl in locs]
                vals = [plsc.load_gather(table_v, [ix]) for ix in idxs]
                for (rr, sl), v in zip(locs, vals):
                    dst[rr, sl] = v
                return carry

            lax.fori_loop(0, rpair, body, 0)
            o_cp[c] = pltpu.async_copy(
                dst, out_hbm.at[pl.ds(c * _RCHUNK, _RCHUNK), pl.ds(col0, cb)],
                sem_o[c % _NBUF])
            if c + nib < nchunks:
                i_cp[c + nib] = pltpu.async_copy(
                    idx_hbm.at[pl.ds((c + nib) * _RCHUNK, _RCHUNK),
                               pl.ds(col0, cb)],
                    idx_b[c % nib], sem_i[c % nib])
        o_cp[nchunks - 2].wait()
        o_cp[nchunks - 1].wait()

    return k(vocab_table, idx_t)


def kernel(token_ids, vocab_table):
    out_t = _gather_call(token_ids.T, vocab_table)
    return out_t.T


# speedup vs baseline: 1.0308x; 1.0308x over previous
import functools

import jax
import jax.numpy as jnp
from jax import lax
from jax.experimental import pallas as pl
from jax.experimental.pallas import tpu as pltpu
from jax.experimental.pallas import tpu_sc as plsc

_VOCAB = 100000
_NC, _NS, _L = 2, 16, 16
_NW = _NC * _NS
_RCHUNK = 40
_NBUF = 2


def _gather_call(idx_t, vocab_table):
    rows, cols = idx_t.shape
    cb = cols // _NW
    nchunks = rows // _RCHUNK
    rpair = _RCHUNK // 2
    nv = cb // _L
    mesh = plsc.VectorSubcoreMesh(core_axis_name="c", subcore_axis_name="s")

    @functools.partial(
        pl.kernel,
        mesh=mesh,
        compiler_params=pltpu.CompilerParams(needs_layout_passes=False),
        out_type=jax.ShapeDtypeStruct((rows, cols), jnp.float32),
        scratch_types=[
            pltpu.VMEM_SHARED((_VOCAB,), jnp.float32),
            pltpu.VMEM((_VOCAB,), jnp.float32),
            pltpu.VMEM((_RCHUNK, cb), jnp.int32),
            pltpu.VMEM((_RCHUNK, cb), jnp.int32),
            pltpu.VMEM((_RCHUNK, cb), jnp.float32),
            pltpu.VMEM((_RCHUNK, cb), jnp.float32),
            pltpu.SemaphoreType.DMA,
            pltpu.SemaphoreType.DMA,
            pltpu.SemaphoreType.DMA,
            pltpu.SemaphoreType.DMA,
            pltpu.SemaphoreType.DMA,
        ],
    )
    def k(table_hbm, idx_hbm, out_hbm, table_sh, table_v, idx_v0, idx_v1,
          out_v0, out_v1, sem_t, sem_i0, sem_i1, sem_o0, sem_o1):
        idx_b = (idx_v0, idx_v1)
        out_b = (out_v0, out_v1)
        sem_i = (sem_i0, sem_i1)
        sem_o = (sem_o0, sem_o1)
        nib = len(idx_b)
        wid = lax.axis_index("s") * _NC + lax.axis_index("c")
        col0 = wid * cb

        i_cp = [None] * nchunks
        o_cp = [None] * nchunks
        for c in range(min(nib, nchunks)):
            i_cp[c] = pltpu.async_copy(
                idx_hbm.at[pl.ds(c * _RCHUNK, _RCHUNK), pl.ds(col0, cb)],
                idx_b[c % nib], sem_i[c % nib])
        @pl.when(lax.axis_index("s") == 0)
        def _load_spmem():
            pltpu.sync_copy(table_hbm, table_sh)

        plsc.subcore_barrier()
        t_cp = pltpu.async_copy(table_sh, table_v, sem_t)
        t_cp.wait()

        for c in range(nchunks):
            i_cp[c].wait()
            if c >= _NBUF:
                o_cp[c - _NBUF].wait()
            src = idx_b[c % nib]
            dst = out_b[c % _NBUF]

            def body(r, carry):
                locs = [(r * 2 + j, pl.ds(v * _L, _L))
                        for j in range(2) for v in range(nv)]
                idxs = [src[rr, sl] for rr, sl in locs]
                vals = [plsc.load_gather(table_v, [ix]) for ix in idxs]
                for (rr, sl), v in zip(locs, vals):
                    dst[rr, sl] = v
                return carry

            lax.fori_loop(0, rpair, body, 0)
            o_cp[c] = pltpu.async_copy(
                dst, out_hbm.at[pl.ds(c * _RCHUNK, _RCHUNK), pl.ds(col0, cb)],
                sem_o[c % _NBUF])
            if c + nib < nchunks:
                i_cp[c + nib] = pltpu.async_copy(
                    idx_hbm.at[pl.ds((c + nib) * _RCHUNK, _RCHUNK),
                               pl.ds(col0, cb)],
                    idx_b[c % nib], sem_i[c % nib])
        o_cp[nchunks - 2].wait()
        o_cp[nchunks - 1].wait()

    return k(vocab_table, idx_t)


def kernel(token_ids, vocab_table):
    out_t = _gather_call(token_ids.T, vocab_table)
    return out_t.T
